# baseline (speedup 1.0000x reference)
import functools

import jax
import jax.numpy as jnp
from jax import lax
from jax.experimental import pallas as pl
from jax.experimental.pallas import tpu as pltpu
from jax.experimental.pallas import tpu_sc as plsc

N = 10000
NP = 10112
PADROW = NP - 1
E = 320000
NT = 32
BS = 128
NB = 81
EPT = NB * BS
EPAD = EPT * NT
RB = NP // 16

_MESH = dict(core_axis_name="c", subcore_axis_name="s")


def _make_agg(D):
    @functools.partial(
        pl.kernel,
        mesh=plsc.VectorSubcoreMesh(**_MESH),
        out_type=[jax.ShapeDtypeStruct((NP, D), jnp.float32),
                  jax.ShapeDtypeStruct((NP, D), jnp.float32)],
        scratch_types=[
            pltpu.VMEM((3, BS), jnp.int32),
            pltpu.VMEM((3, BS), jnp.int32),
            pltpu.VMEM((BS, D), jnp.float32),
            pltpu.VMEM((BS, D), jnp.float32),
            pltpu.VMEM((BS, D), jnp.float32),
            pltpu.VMEM_SHARED((NP, D), jnp.float32),
        ] + [pltpu.SemaphoreType.DMA] * 12,
    )
    def agg(src_hbm, dst_hbm, t_hbm, zeros_hbm, out0, out1,
            sidx, didx, b0, b1, b2, acc,
            gs0, gs1, gs2, ss0, ss1, ss2,
            is0, is1, is2, id0, id1, id2):
        bufs = (b0, b1, b2)
        gsem = (gs0, gs1, gs2)
        ssem = (ss0, ss1, ss2)
        isem_s = (is0, is1, is2)
        isem_d = (id0, id1, id2)
        c = lax.axis_index("c")
        s = lax.axis_index("s")
        w = c * 16 + s
        rs = s * RB

        def pf_src(u, jj, sem):
            pltpu.async_copy(src_hbm.at[w, jj], sidx.at[u], sem)

        def pf_src_wait(u, sem):
            pltpu.make_async_copy(src_hbm.at[w, 0], sidx.at[u], sem).wait()

        def pf_dst(u, jj, sem):
            pltpu.async_copy(dst_hbm.at[w, jj], didx.at[u], sem)

        def pf_dst_wait(u, sem):
            pltpu.make_async_copy(dst_hbm.at[w, 0], didx.at[u], sem).wait()

        def gstart(u, sem):
            pltpu.async_copy(t_hbm.at[sidx.at[u]], bufs[u], sem)

        def gwait(u, sem):
            pltpu.make_async_copy(t_hbm.at[sidx.at[u]], bufs[u], sem).wait()

        def sstart(u, sem):
            pltpu.async_copy(bufs[u], acc.at[didx.at[u]], sem, add=True)

        def swait(u, sem):
            pltpu.make_async_copy(bufs[u], acc.at[didx.at[u]], sem).wait()

        @pl.when(c == 0)
        def _():
            pltpu.sync_copy(t_hbm.at[pl.ds(rs, RB)], acc.at[pl.ds(rs, RB)])

        @pl.when(c != 0)
        def _():
            pltpu.sync_copy(zeros_hbm.at[pl.ds(rs, RB)], acc.at[pl.ds(rs, RB)])

        plsc.subcore_barrier()

        pf_src(0, 0, isem_s[0])
        pf_dst(0, 0, isem_d[0])
        pf_src_wait(0, isem_s[0])
        gstart(0, gsem[0])
        pf_src(1, 1, isem_s[1])
        pf_dst(1, 1, isem_d[1])
        pf_src_wait(1, isem_s[1])
        gstart(1, gsem[1])
        pf_src(2, 2, isem_s[2])
        pf_dst(2, 2, isem_d[2])
        pf_src_wait(2, isem_s[2])
        gstart(2, gsem[2])
        gwait(0, gsem[0])
        pf_src(0, 3, isem_s[0])
        pf_dst_wait(0, isem_d[0])
        sstart(0, ssem[0])

        def steady(u, jj):
            v = (u + 1) % 3
            swait(u, ssem[u])
            pf_dst(u, jj, isem_d[u])
            pf_src_wait(u, isem_s[u])
            gstart(u, gsem[u])
            gwait(v, gsem[v])
            pf_src(v, lax.rem(jj + 1, NB), isem_s[v])
            pf_dst_wait(v, isem_d[v])
            sstart(v, ssem[v])

        def body(i2, carry):
            j0 = 3 * i2 + 3
            steady(0, j0)
            steady(1, j0 + 1)
            steady(2, j0 + 2)
            return carry

        lax.fori_loop(0, (NB - 3) // 3, body, 0)

        gwait(1, gsem[1])
        pf_dst_wait(1, isem_d[1])
        sstart(1, ssem[1])
        gwait(2, gsem[2])
        pf_dst_wait(2, isem_d[2])
        sstart(2, ssem[2])
        swait(0, ssem[0])
        swait(1, ssem[1])
        swait(2, ssem[2])
        pf_src_wait(0, isem_s[0])
        plsc.subcore_barrier()

        @pl.when(c == 0)
        def _():
            pltpu.sync_copy(acc.at[pl.ds(rs, RB)], out0.at[pl.ds(rs, RB)])

        @pl.when(c != 0)
        def _():
            pltpu.sync_copy(acc.at[pl.ds(rs, RB)], out1.at[pl.ds(rs, RB)])

    return agg


_agg128 = _make_agg(128)


@functools.partial(
    pl.kernel,
    mesh=plsc.VectorSubcoreMesh(**_MESH),
    out_type=[jax.ShapeDtypeStruct((NP, 128), jnp.float32),
              jax.ShapeDtypeStruct((NP, 128), jnp.float32)],
    scratch_types=[
        pltpu.VMEM((6, BS), jnp.int32),
        pltpu.VMEM((BS, 128), jnp.float32),
        pltpu.VMEM_SHARED((NP, 128), jnp.float32),
    ] + [pltpu.SemaphoreType.DMA] * 6,
)
def _deg(dst_hbm, ones_hbm, zeros_hbm, out0, out1,
         didx, onesb, acc, ss0, ss1, ss2, id0, id1, id2):
    ssem = (ss0, ss1, ss2)
    isem = (id0, id1, id2)
    c = lax.axis_index("c")
    s = lax.axis_index("s")
    w = c * 16 + s
    rs = s * RB

    def pf(jj, sem):
        pltpu.async_copy(dst_hbm.at[w, jj], didx.at[lax.rem(jj, 6)], sem)

    def pf_wait(sem):
        pltpu.make_async_copy(dst_hbm.at[w, 0], didx.at[0], sem).wait()

    def sstart(jj, sem):
        pltpu.async_copy(onesb, acc.at[didx.at[lax.rem(jj, 6)]], sem, add=True)

    def swait(sem):
        pltpu.make_async_copy(onesb, acc.at[didx.at[0]], sem).wait()

    pltpu.sync_copy(ones_hbm.at[pl.ds(0, BS)], onesb)

    @pl.when(c == 0)
    def _():
        pltpu.sync_copy(ones_hbm.at[pl.ds(rs, RB)], acc.at[pl.ds(rs, RB)])

    @pl.when(c != 0)
    def _():
        pltpu.sync_copy(zeros_hbm.at[pl.ds(rs, RB)], acc.at[pl.ds(rs, RB)])

    plsc.subcore_barrier()

    for u in range(3):
        pf(u, isem[u])
    for u in range(3):
        pf_wait(isem[u])
        sstart(u, ssem[u])
        pf(u + 3, isem[u])

    def steady(u, jj):
        swait(ssem[u])
        pf_wait(isem[u])
        sstart(jj, ssem[u])
        pf(lax.rem(jj + 3, NB), isem[u])

    def body(i2, carry):
        j0 = 3 * i2 + 3
        steady(0, j0)
        steady(1, j0 + 1)
        steady(2, j0 + 2)
        return carry

    lax.fori_loop(0, (NB - 3) // 3, body, 0)

    for u in range(3):
        swait(ssem[u])
        pf_wait(isem[u])
    plsc.subcore_barrier()

    @pl.when(c == 0)
    def _():
        pltpu.sync_copy(acc.at[pl.ds(rs, RB)], out0.at[pl.ds(rs, RB)])

    @pl.when(c != 0)
    def _():
        pltpu.sync_copy(acc.at[pl.ds(rs, RB)], out1.at[pl.ds(rs, RB)])


def _first_body(x_ref, w_ref, d0_ref, d1_ref, t_ref, dinv_ref):
    deg = d0_ref[...] + d1_ref[...]
    dinv = lax.rsqrt(deg)
    mm = lax.dot_general(x_ref[...], w_ref[...], (((1,), (0,)), ((), ())),
                         precision=lax.Precision.HIGHEST,
                         preferred_element_type=jnp.float32)
    t_ref[...] = mm * dinv
    dinv_ref[...] = dinv


def _first(xp, W1, d0, d1):
    return pl.pallas_call(
        _first_body,
        grid=(16,),
        in_specs=[
            pl.BlockSpec((RB, 128), lambda i: (i, 0)),
            pl.BlockSpec((128, 128), lambda i: (0, 0)),
            pl.BlockSpec((RB, 1), lambda i: (i, 0)),
            pl.BlockSpec((RB, 1), lambda i: (i, 0)),
        ],
        out_specs=[
            pl.BlockSpec((RB, 128), lambda i: (i, 0)),
            pl.BlockSpec((RB, 1), lambda i: (i, 0)),
        ],
        out_shape=[
            jax.ShapeDtypeStruct((NP, 128), jnp.float32),
            jax.ShapeDtypeStruct((NP, 1), jnp.float32),
        ],
    )(xp, W1, d0, d1)


def _mid_body(a0_ref, a1_ref, dinv_ref, b_ref, w_ref, t_ref):
    dinv = dinv_ref[...]
    act = jnp.maximum((a0_ref[...] + a1_ref[...]) * dinv + b_ref[...], 0.0)
    mm = lax.dot_general(act, w_ref[...], (((1,), (0,)), ((), ())),
                         precision=lax.Precision.HIGHEST,
                         preferred_element_type=jnp.float32)
    t_ref[...] = mm * dinv


def _mid(a0, a1, dinv, b, W, d_out):
    return pl.pallas_call(
        _mid_body,
        grid=(16,),
        in_specs=[
            pl.BlockSpec((RB, 128), lambda i: (i, 0)),
            pl.BlockSpec((RB, 128), lambda i: (i, 0)),
            pl.BlockSpec((RB, 1), lambda i: (i, 0)),
            pl.BlockSpec((128,), lambda i: (0,)),
            pl.BlockSpec((128, d_out), lambda i: (0, 0)),
        ],
        out_specs=pl.BlockSpec((RB, d_out), lambda i: (i, 0)),
        out_shape=jax.ShapeDtypeStruct((NP, d_out), jnp.float32),
    )(a0, a1, dinv, b, W)


def _final_body(a0_ref, a1_ref, dinv_ref, b_ref, o_ref):
    a = a0_ref[...] + a1_ref[...]
    o_ref[...] = a[:, :40] * dinv_ref[...] + b_ref[...]


def _final(a0, a1, dinv, b3):
    return pl.pallas_call(
        _final_body,
        grid=(25,),
        in_specs=[
            pl.BlockSpec((400, 128), lambda i: (i, 0)),
            pl.BlockSpec((400, 128), lambda i: (i, 0)),
            pl.BlockSpec((400, 1), lambda i: (i, 0)),
            pl.BlockSpec((40,), lambda i: (0,)),
        ],
        out_specs=pl.BlockSpec((400, 40), lambda i: (i, 0)),
        out_shape=jax.ShapeDtypeStruct((N, 40), jnp.float32),
    )(a0, a1, dinv, b3)


def kernel(x, edge_index, W1, b1, W2, b2, W3, b3):
    x = x.astype(jnp.float32)
    src = edge_index[0].astype(jnp.int32)
    dst = edge_index[1].astype(jnp.int32)
    pad = jnp.full((EPAD - E,), PADROW, jnp.int32)
    src3 = jnp.concatenate([src, pad]).reshape(NT, NB, BS)
    dst3 = jnp.concatenate([dst, pad]).reshape(NT, NB, BS)

    xp = jnp.pad(x, ((0, NP - N), (0, 0)))
    zeros128 = jnp.zeros((NP, 128), jnp.float32)
    ones128 = jnp.ones((NP, 128), jnp.float32)
    W3p = jnp.pad(W3.astype(jnp.float32), ((0, 0), (0, 128 - 40)))

    g0, g1 = _agg128(src3, dst3, ones128, zeros128)
    t1, dinv = _first(xp, W1, g0[:, :1], g1[:, :1])
    a0, a1 = _agg128(src3, dst3, t1, zeros128)
    t2 = _mid(a0, a1, dinv, b1, W2, 128)
    a0, a1 = _agg128(src3, dst3, t2, zeros128)
    t3 = _mid(a0, a1, dinv, b2, W3p, 128)
    a0, a1 = _agg128(src3, dst3, t3, zeros128)
    return _final(a0, a1, dinv, b3)


# speedup vs baseline: 1.0827x; 1.0827x over previous
import functools

import jax
import jax.numpy as jnp
from jax import lax
from jax.experimental import pallas as pl
from jax.experimental.pallas import tpu as pltpu
from jax.experimental.pallas import tpu_sc as plsc

N = 10000
NP = 10112
PADROW = NP - 1
E = 320000
NT = 32
BS = 128
NB = 81
EPT = NB * BS
EPAD = EPT * NT
RB = NP // 16

_MESH = dict(core_axis_name="c", subcore_axis_name="s")


def _make_agg(D):
    @functools.partial(
        pl.kernel,
        mesh=plsc.VectorSubcoreMesh(**_MESH),
        out_type=[jax.ShapeDtypeStruct((NP, D), jnp.float32),
                  jax.ShapeDtypeStruct((NP, D), jnp.float32)],
        scratch_types=[
            pltpu.VMEM((NB, BS), jnp.int32),
            pltpu.VMEM((NB, BS), jnp.int32),
            pltpu.VMEM((BS, D), jnp.float32),
            pltpu.VMEM_SHARED((NP, D), jnp.float32),
            pltpu.SemaphoreType.DMA,
        ],
    )
    def agg(src_hbm, dst_hbm, t_hbm, zeros_hbm, out0, out1,
            idx_s, idx_d, buf0, acc, sem0):
        c = lax.axis_index("c")
        s = lax.axis_index("s")
        w = c * 16 + s
        pltpu.sync_copy(src_hbm.at[w], idx_s)
        pltpu.sync_copy(dst_hbm.at[w], idx_d)
        rs = s * RB

        @pl.when(c == 0)
        def _():
            pltpu.sync_copy(t_hbm.at[pl.ds(rs, RB)], acc.at[pl.ds(rs, RB)])

        @pl.when(c != 0)
        def _():
            pltpu.sync_copy(zeros_hbm.at[pl.ds(rs, RB)], acc.at[pl.ds(rs, RB)])

        plsc.subcore_barrier()

        def body(j, carry):
            pltpu.async_copy(t_hbm.at[idx_s.at[j]], buf0, sem0).wait()
            pltpu.sync_copy(buf0, acc.at[idx_d.at[j]], add=True)
            return carry

        lax.fori_loop(0, NB, body, 0)
        plsc.subcore_barrier()

        @pl.when(c == 0)
        def _():
            pltpu.sync_copy(acc.at[pl.ds(rs, RB)], out0.at[pl.ds(rs, RB)])

        @pl.when(c != 0)
        def _():
            pltpu.sync_copy(acc.at[pl.ds(rs, RB)], out1.at[pl.ds(rs, RB)])

    return agg


_agg128 = _make_agg(128)

DW = 8


@functools.partial(
    pl.kernel,
    mesh=plsc.VectorSubcoreMesh(**_MESH),
    out_type=[jax.ShapeDtypeStruct((NP, DW), jnp.float32),
              jax.ShapeDtypeStruct((NP, DW), jnp.float32)],
    scratch_types=[
        pltpu.VMEM((NB, BS), jnp.int32),
        pltpu.VMEM((BS, DW), jnp.float32),
        pltpu.VMEM_SHARED((NP, DW), jnp.float32),
        pltpu.SemaphoreType.DMA,
    ],
)
def _deg(dst_hbm, ones_hbm, zeros_hbm, out0, out1, idx_d, onesb, acc, sem0):
    c = lax.axis_index("c")
    s = lax.axis_index("s")
    w = c * 16 + s
    rs = s * RB
    pltpu.sync_copy(dst_hbm.at[w], idx_d)
    pltpu.sync_copy(ones_hbm.at[pl.ds(0, BS)], onesb)

    @pl.when(c == 0)
    def _():
        pltpu.sync_copy(ones_hbm.at[pl.ds(rs, RB)], acc.at[pl.ds(rs, RB)])

    @pl.when(c != 0)
    def _():
        pltpu.sync_copy(zeros_hbm.at[pl.ds(rs, RB)], acc.at[pl.ds(rs, RB)])

    plsc.subcore_barrier()

    def body(j, carry):
        pltpu.sync_copy(onesb, acc.at[idx_d.at[j]], add=True)
        return carry

    lax.fori_loop(0, NB, body, 0)
    plsc.subcore_barrier()

    @pl.when(c == 0)
    def _():
        pltpu.sync_copy(acc.at[pl.ds(rs, RB)], out0.at[pl.ds(rs, RB)])

    @pl.when(c != 0)
    def _():
        pltpu.sync_copy(acc.at[pl.ds(rs, RB)], out1.at[pl.ds(rs, RB)])


def _first_body(x_ref, w_ref, d0_ref, d1_ref, t_ref, dinv_ref):
    deg = d0_ref[...] + d1_ref[...]
    dinv = lax.rsqrt(deg)
    mm = lax.dot_general(x_ref[...], w_ref[...], (((1,), (0,)), ((), ())),
                         precision=lax.Precision.HIGHEST,
                         preferred_element_type=jnp.float32)
    t_ref[...] = mm * dinv
    dinv_ref[...] = dinv


def _first(xp, W1, d0, d1):
    return pl.pallas_call(
        _first_body,
        grid=(16,),
        in_specs=[
            pl.BlockSpec((RB, 128), lambda i: (i, 0)),
            pl.BlockSpec((128, 128), lambda i: (0, 0)),
            pl.BlockSpec((RB, 1), lambda i: (i, 0)),
            pl.BlockSpec((RB, 1), lambda i: (i, 0)),
        ],
        out_specs=[
            pl.BlockSpec((RB, 128), lambda i: (i, 0)),
            pl.BlockSpec((RB, 1), lambda i: (i, 0)),
        ],
        out_shape=[
            jax.ShapeDtypeStruct((NP, 128), jnp.float32),
            jax.ShapeDtypeStruct((NP, 1), jnp.float32),
        ],
    )(xp, W1, d0, d1)


def _mid_body(a0_ref, a1_ref, dinv_ref, b_ref, w_ref, t_ref):
    dinv = dinv_ref[...]
    act = jnp.maximum((a0_ref[...] + a1_ref[...]) * dinv + b_ref[...], 0.0)
    mm = lax.dot_general(act, w_ref[...], (((1,), (0,)), ((), ())),
                         precision=lax.Precision.HIGHEST,
                         preferred_element_type=jnp.float32)
    t_ref[...] = mm * dinv


def _mid(a0, a1, dinv, b, W, d_out):
    return pl.pallas_call(
        _mid_body,
        grid=(16,),
        in_specs=[
            pl.BlockSpec((RB, 128), lambda i: (i, 0)),
            pl.BlockSpec((RB, 128), lambda i: (i, 0)),
            pl.BlockSpec((RB, 1), lambda i: (i, 0)),
            pl.BlockSpec((128,), lambda i: (0,)),
            pl.BlockSpec((128, d_out), lambda i: (0, 0)),
        ],
        out_specs=pl.BlockSpec((RB, d_out), lambda i: (i, 0)),
        out_shape=jax.ShapeDtypeStruct((NP, d_out), jnp.float32),
    )(a0, a1, dinv, b, W)


def _final_body(a0_ref, a1_ref, dinv_ref, b_ref, o_ref):
    a = a0_ref[...] + a1_ref[...]
    o_ref[...] = a[:, :40] * dinv_ref[...] + b_ref[...]


def _final(a0, a1, dinv, b3):
    return pl.pallas_call(
        _final_body,
        grid=(25,),
        in_specs=[
            pl.BlockSpec((400, 128), lambda i: (i, 0)),
            pl.BlockSpec((400, 128), lambda i: (i, 0)),
            pl.BlockSpec((400, 1), lambda i: (i, 0)),
            pl.BlockSpec((40,), lambda i: (0,)),
        ],
        out_specs=pl.BlockSpec((400, 40), lambda i: (i, 0)),
        out_shape=jax.ShapeDtypeStruct((N, 40), jnp.float32),
    )(a0, a1, dinv, b3)


def kernel(x, edge_index, W1, b1, W2, b2, W3, b3):
    x = x.astype(jnp.float32)
    src = edge_index[0].astype(jnp.int32)
    dst = edge_index[1].astype(jnp.int32)
    pad = jnp.full((EPAD - E,), PADROW, jnp.int32)
    src3 = jnp.concatenate([src, pad]).reshape(NT, NB, BS)
    dst3 = jnp.concatenate([dst, pad]).reshape(NT, NB, BS)

    xp = jnp.pad(x, ((0, NP - N), (0, 0)))
    zeros128 = jnp.zeros((NP, 128), jnp.float32)
    ones8 = jnp.ones((NP, DW), jnp.float32)
    zeros8 = jnp.zeros((NP, DW), jnp.float32)
    W3p = jnp.pad(W3.astype(jnp.float32), ((0, 0), (0, 128 - 40)))

    g0, g1 = _deg(dst3, ones8, zeros8)
    t1, dinv = _first(xp, W1, g0[:, :1], g1[:, :1])
    a0, a1 = _agg128(src3, dst3, t1, zeros128)
    t2 = _mid(a0, a1, dinv, b1, W2, 128)
    a0, a1 = _agg128(src3, dst3, t2, zeros128)
    t3 = _mid(a0, a1, dinv, b2, W3p, 128)
    a0, a1 = _agg128(src3, dst3, t3, zeros128)
    return _final(a0, a1, dinv, b3)


# speedup vs baseline: 1.2579x; 1.1618x over previous
import functools

import jax
import jax.numpy as jnp
from jax import lax
from jax.experimental import pallas as pl
from jax.experimental.pallas import tpu as pltpu
from jax.experimental.pallas import tpu_sc as plsc

N = 10000
NP = 10112
PADROW = NP - 1
E = 320000
NT = 32
BS = 128
NB = 81
EPT = NB * BS
EPAD = EPT * NT
RB = NP // 16

_MESH = dict(core_axis_name="c", subcore_axis_name="s")


def _make_agg(D):
    @functools.partial(
        pl.kernel,
        mesh=plsc.VectorSubcoreMesh(**_MESH),
        out_type=[jax.ShapeDtypeStruct((NP, D), jnp.float32),
                  jax.ShapeDtypeStruct((NP, D), jnp.float32)],
        scratch_types=[
            pltpu.VMEM((NB, BS), jnp.int32),
            pltpu.VMEM((NB, BS), jnp.int32),
            pltpu.VMEM((BS, D), jnp.float32),
            pltpu.VMEM_SHARED((NP, D), jnp.float32),
            pltpu.SemaphoreType.DMA,
        ],
    )
    def agg(src_hbm, dst_hbm, t_hbm, zeros_hbm, out0, out1,
            idx_s, idx_d, buf0, acc, sem0):
        c = lax.axis_index("c")
        s = lax.axis_index("s")
        w = c * 16 + s
        pltpu.sync_copy(src_hbm.at[w], idx_s)
        pltpu.sync_copy(dst_hbm.at[w], idx_d)
        rs = s * RB

        @pl.when(c == 0)
        def _():
            pltpu.sync_copy(t_hbm.at[pl.ds(rs, RB)], acc.at[pl.ds(rs, RB)])

        @pl.when(c != 0)
        def _():
            pltpu.sync_copy(zeros_hbm.at[pl.ds(rs, RB)], acc.at[pl.ds(rs, RB)])

        plsc.subcore_barrier()

        def body(j, carry):
            pltpu.async_copy(t_hbm.at[idx_s.at[j]], buf0, sem0).wait()
            pltpu.sync_copy(buf0, acc.at[idx_d.at[j]], add=True)
            return carry

        lax.fori_loop(0, NB, body, 0)
        plsc.subcore_barrier()

        @pl.when(c == 0)
        def _():
            pltpu.sync_copy(acc.at[pl.ds(rs, RB)], out0.at[pl.ds(rs, RB)])

        @pl.when(c != 0)
        def _():
            pltpu.sync_copy(acc.at[pl.ds(rs, RB)], out1.at[pl.ds(rs, RB)])

    return agg


_agg128 = _make_agg(128)

DW = 128


@functools.partial(
    pl.kernel,
    mesh=plsc.VectorSubcoreMesh(**_MESH),
    out_type=[jax.ShapeDtypeStruct((NP, DW), jnp.float32),
              jax.ShapeDtypeStruct((NP, DW), jnp.float32)],
    scratch_types=[
        pltpu.VMEM((NB, BS), jnp.int32),
        pltpu.VMEM((BS, DW), jnp.float32),
        pltpu.VMEM_SHARED((NP, DW), jnp.float32),
        pltpu.SemaphoreType.DMA,
    ],
)
def _deg(dst_hbm, ones_hbm, zeros_hbm, out0, out1, idx_d, onesb, acc, sem0):
    c = lax.axis_index("c")
    s = lax.axis_index("s")
    w = c * 16 + s
    rs = s * RB
    pltpu.sync_copy(dst_hbm.at[w], idx_d)
    pltpu.sync_copy(ones_hbm.at[pl.ds(0, BS)], onesb)

    @pl.when(c == 0)
    def _():
        pltpu.sync_copy(ones_hbm.at[pl.ds(rs, RB)], acc.at[pl.ds(rs, RB)])

    @pl.when(c != 0)
    def _():
        pltpu.sync_copy(zeros_hbm.at[pl.ds(rs, RB)], acc.at[pl.ds(rs, RB)])

    plsc.subcore_barrier()

    def body(j, carry):
        pltpu.sync_copy(onesb, acc.at[idx_d.at[j]], add=True)
        return carry

    lax.fori_loop(0, NB, body, 0)
    plsc.subcore_barrier()

    @pl.when(c == 0)
    def _():
        pltpu.sync_copy(acc.at[pl.ds(rs, RB)], out0.at[pl.ds(rs, RB)])

    @pl.when(c != 0)
    def _():
        pltpu.sync_copy(acc.at[pl.ds(rs, RB)], out1.at[pl.ds(rs, RB)])


def _first_body(x_ref, w_ref, d0_ref, d1_ref, t_ref, dinv_ref):
    deg = d0_ref[...] + d1_ref[...]
    dinv = lax.rsqrt(deg)
    mm = lax.dot_general(x_ref[...], w_ref[...], (((1,), (0,)), ((), ())),
                         precision=lax.Precision.HIGHEST,
                         preferred_element_type=jnp.float32)
    t_ref[...] = mm * dinv
    dinv_ref[...] = dinv


def _first(xp, W1, d0, d1):
    return pl.pallas_call(
        _first_body,
        grid=(16,),
        in_specs=[
            pl.BlockSpec((RB, 128), lambda i: (i, 0)),
            pl.BlockSpec((128, 128), lambda i: (0, 0)),
            pl.BlockSpec((RB, 1), lambda i: (i, 0)),
            pl.BlockSpec((RB, 1), lambda i: (i, 0)),
        ],
        out_specs=[
            pl.BlockSpec((RB, 128), lambda i: (i, 0)),
            pl.BlockSpec((RB, 1), lambda i: (i, 0)),
        ],
        out_shape=[
            jax.ShapeDtypeStruct((NP, 128), jnp.float32),
            jax.ShapeDtypeStruct((NP, 1), jnp.float32),
        ],
    )(xp, W1, d0, d1)


def _mid_body(a0_ref, a1_ref, dinv_ref, b_ref, w_ref, t_ref):
    dinv = dinv_ref[...]
    act = jnp.maximum((a0_ref[...] + a1_ref[...]) * dinv + b_ref[...], 0.0)
    mm = lax.dot_general(act, w_ref[...], (((1,), (0,)), ((), ())),
                         precision=lax.Precision.HIGHEST,
                         preferred_element_type=jnp.float32)
    t_ref[...] = mm * dinv


def _mid(a0, a1, dinv, b, W, d_out):
    return pl.pallas_call(
        _mid_body,
        grid=(16,),
        in_specs=[
            pl.BlockSpec((RB, 128), lambda i: (i, 0)),
            pl.BlockSpec((RB, 128), lambda i: (i, 0)),
            pl.BlockSpec((RB, 1), lambda i: (i, 0)),
            pl.BlockSpec((128,), lambda i: (0,)),
            pl.BlockSpec((128, d_out), lambda i: (0, 0)),
        ],
        out_specs=pl.BlockSpec((RB, d_out), lambda i: (i, 0)),
        out_shape=jax.ShapeDtypeStruct((NP, d_out), jnp.float32),
    )(a0, a1, dinv, b, W)


def _final_body(a0_ref, a1_ref, dinv_ref, b_ref, o_ref):
    a = a0_ref[...] + a1_ref[...]
    o_ref[...] = a[:, :40] * dinv_ref[...] + b_ref[...]


def _final(a0, a1, dinv, b3):
    return pl.pallas_call(
        _final_body,
        grid=(25,),
        in_specs=[
            pl.BlockSpec((400, 128), lambda i: (i, 0)),
            pl.BlockSpec((400, 128), lambda i: (i, 0)),
            pl.BlockSpec((400, 1), lambda i: (i, 0)),
            pl.BlockSpec((40,), lambda i: (0,)),
        ],
        out_specs=pl.BlockSpec((400, 40), lambda i: (i, 0)),
        out_shape=jax.ShapeDtypeStruct((N, 40), jnp.float32),
    )(a0, a1, dinv, b3)


def kernel(x, edge_index, W1, b1, W2, b2, W3, b3):
    x = x.astype(jnp.float32)
    src = edge_index[0].astype(jnp.int32)
    dst = edge_index[1].astype(jnp.int32)
    pad = jnp.full((EPAD - E,), PADROW, jnp.int32)
    src3 = jnp.concatenate([src, pad]).reshape(NT, NB, BS)
    dst3 = jnp.concatenate([dst, pad]).reshape(NT, NB, BS)

    xp = jnp.pad(x, ((0, NP - N), (0, 0)))
    zeros128 = jnp.zeros((NP, 128), jnp.float32)
    ones8 = jnp.ones((NP, DW), jnp.float32)
    zeros8 = jnp.zeros((NP, DW), jnp.float32)
    W3p = jnp.pad(W3.astype(jnp.float32), ((0, 0), (0, 128 - 40)))

    g0, g1 = _deg(dst3, ones8, zeros8)
    t1, dinv = _first(xp, W1, g0[:, :1], g1[:, :1])
    a0, a1 = _agg128(src3, dst3, t1, zeros128)
    t2 = _mid(a0, a1, dinv, b1, W2, 128)
    a0, a1 = _agg128(src3, dst3, t2, zeros128)
    t3 = _mid(a0, a1, dinv, b2, W3p, 128)
    a0, a1 = _agg128(src3, dst3, t3, zeros128)
    return _final(a0, a1, dinv, b3)


# speedup vs baseline: 1.3652x; 1.0853x over previous
import functools

import jax
import jax.numpy as jnp
from jax import lax
from jax.experimental import pallas as pl
from jax.experimental.pallas import tpu as pltpu
from jax.experimental.pallas import tpu_sc as plsc

N = 10000
NP = 10112
PADROW = NP - 1
E = 320000
NT = 32
BS = 128
NB = 80
EPT = NB * BS
EPAD = EPT * NT
RB = NP // 16

_MESH = dict(core_axis_name="c", subcore_axis_name="s")


def _make_agg(D):
    @functools.partial(
        pl.kernel,
        mesh=plsc.VectorSubcoreMesh(**_MESH),
        out_type=[jax.ShapeDtypeStruct((NP, D), jnp.float32),
                  jax.ShapeDtypeStruct((NP, D), jnp.float32)],
        scratch_types=[
            pltpu.VMEM((NB, BS), jnp.int32),
            pltpu.VMEM((NB, BS), jnp.int32),
            pltpu.VMEM((BS, D), jnp.float32),
            pltpu.VMEM_SHARED((NP, D), jnp.float32),
            pltpu.SemaphoreType.DMA,
        ],
    )
    def agg(src_hbm, dst_hbm, t_hbm, zeros_hbm, out0, out1,
            idx_s, idx_d, buf0, acc, sem0):
        c = lax.axis_index("c")
        s = lax.axis_index("s")
        w = c * 16 + s
        pltpu.sync_copy(src_hbm.at[w], idx_s)
        pltpu.sync_copy(dst_hbm.at[w], idx_d)
        rs = s * RB

        @pl.when(c == 0)
        def _():
            pltpu.sync_copy(t_hbm.at[pl.ds(rs, RB)], acc.at[pl.ds(rs, RB)])

        @pl.when(c != 0)
        def _():
            pltpu.sync_copy(zeros_hbm.at[pl.ds(rs, RB)], acc.at[pl.ds(rs, RB)])

        plsc.subcore_barrier()

        def body(j, carry):
            pltpu.async_copy(t_hbm.at[idx_s.at[j]], buf0, sem0).wait()
            pltpu.sync_copy(buf0, acc.at[idx_d.at[j]], add=True)
            return carry

        lax.fori_loop(0, NB, body, 0)
        plsc.subcore_barrier()

        @pl.when(c == 0)
        def _():
            pltpu.sync_copy(acc.at[pl.ds(rs, RB)], out0.at[pl.ds(rs, RB)])

        @pl.when(c != 0)
        def _():
            pltpu.sync_copy(acc.at[pl.ds(rs, RB)], out1.at[pl.ds(rs, RB)])

    return agg


_agg128 = _make_agg(128)


def _first_body(x_ref, w_ref, d0_ref, d1_ref, t_ref, dinv_ref):
    deg = d0_ref[...] + d1_ref[...]
    dinv = lax.rsqrt(deg)
    mm = lax.dot_general(x_ref[...], w_ref[...], (((1,), (0,)), ((), ())),
                         precision=lax.Precision.HIGHEST,
                         preferred_element_type=jnp.float32)
    t_ref[...] = mm * dinv
    dinv_ref[...] = dinv


def _first(xp, W1, d0, d1):
    return pl.pallas_call(
        _first_body,
        grid=(16,),
        in_specs=[
            pl.BlockSpec((RB, 128), lambda i: (i, 0)),
            pl.BlockSpec((128, 128), lambda i: (0, 0)),
            pl.BlockSpec((RB, 1), lambda i: (i, 0)),
            pl.BlockSpec((RB, 1), lambda i: (i, 0)),
        ],
        out_specs=[
            pl.BlockSpec((RB, 128), lambda i: (i, 0)),
            pl.BlockSpec((RB, 1), lambda i: (i, 0)),
        ],
        out_shape=[
            jax.ShapeDtypeStruct((NP, 128), jnp.float32),
            jax.ShapeDtypeStruct((NP, 1), jnp.float32),
        ],
    )(xp, W1, d0, d1)


def _mid_body(a0_ref, a1_ref, dinv_ref, b_ref, w_ref, t_ref):
    dinv = dinv_ref[...]
    act = jnp.maximum((a0_ref[...] + a1_ref[...]) * dinv + b_ref[...], 0.0)
    mm = lax.dot_general(act, w_ref[...], (((1,), (0,)), ((), ())),
                         precision=lax.Precision.HIGHEST,
                         preferred_element_type=jnp.float32)
    t_ref[...] = mm * dinv


def _mid(a0, a1, dinv, b, W, d_out):
    return pl.pallas_call(
        _mid_body,
        grid=(16,),
        in_specs=[
            pl.BlockSpec((RB, 128), lambda i: (i, 0)),
            pl.BlockSpec((RB, 128), lambda i: (i, 0)),
            pl.BlockSpec((RB, 1), lambda i: (i, 0)),
            pl.BlockSpec((128,), lambda i: (0,)),
            pl.BlockSpec((128, d_out), lambda i: (0, 0)),
        ],
        out_specs=pl.BlockSpec((RB, d_out), lambda i: (i, 0)),
        out_shape=jax.ShapeDtypeStruct((NP, d_out), jnp.float32),
    )(a0, a1, dinv, b, W)


def _final_body(a0_ref, a1_ref, dinv_ref, b_ref, o_ref):
    a = a0_ref[...] + a1_ref[...]
    o_ref[...] = a[:, :40] * dinv_ref[...] + b_ref[...]


def _final(a0, a1, dinv, b3):
    return pl.pallas_call(
        _final_body,
        grid=(25,),
        in_specs=[
            pl.BlockSpec((400, 128), lambda i: (i, 0)),
            pl.BlockSpec((400, 128), lambda i: (i, 0)),
            pl.BlockSpec((400, 1), lambda i: (i, 0)),
            pl.BlockSpec((40,), lambda i: (0,)),
        ],
        out_specs=pl.BlockSpec((400, 40), lambda i: (i, 0)),
        out_shape=jax.ShapeDtypeStruct((N, 40), jnp.float32),
    )(a0, a1, dinv, b3)


def kernel(x, edge_index, W1, b1, W2, b2, W3, b3):
    x = x.astype(jnp.float32)
    src = edge_index[0].astype(jnp.int32)
    dst = edge_index[1].astype(jnp.int32)
    pad = jnp.full((EPAD - E,), PADROW, jnp.int32)
    src3 = jnp.concatenate([src, pad]).reshape(NT, NB, BS)
    dst3 = jnp.concatenate([dst, pad]).reshape(NT, NB, BS)

    xp = jnp.pad(x, ((0, NP - N), (0, 0)))
    zeros128 = jnp.zeros((NP, 128), jnp.float32)
    ones128 = jnp.ones((NP, 128), jnp.float32)
    W3p = jnp.pad(W3.astype(jnp.float32), ((0, 0), (0, 128 - 40)))

    g0, g1 = _agg128(src3, dst3, ones128, zeros128)
    t1, dinv = _first(xp, W1, g0[:, :1], g1[:, :1])
    a0, a1 = _agg128(src3, dst3, t1, zeros128)
    t2 = _mid(a0, a1, dinv, b1, W2, 128)
    a0, a1 = _agg128(src3, dst3, t2, zeros128)
    t3 = _mid(a0, a1, dinv, b2, W3p, 128)
    a0, a1 = _agg128(src3, dst3, t3, zeros128)
    return _final(a0, a1, dinv, b3)


# speedup vs baseline: 1.4788x; 1.0832x over previous
import functools

import jax
import jax.numpy as jnp
from jax import lax
from jax.experimental import pallas as pl
from jax.experimental.pallas import tpu as pltpu
from jax.experimental.pallas import tpu_sc as plsc

N = 10000
NP = 10112
PADROW = NP - 1
E = 320000
NT = 32
BS = 128
NB = 80
EPT = NB * BS
EPAD = EPT * NT
RB = NP // 16

_MESH = dict(core_axis_name="c", subcore_axis_name="s")


def _make_agg(D):
    @functools.partial(
        pl.kernel,
        mesh=plsc.VectorSubcoreMesh(**_MESH),
        out_type=[jax.ShapeDtypeStruct((NP, D), jnp.float32),
                  jax.ShapeDtypeStruct((NP, D), jnp.float32)],
        scratch_types=[
            pltpu.VMEM((NB, BS), jnp.int32),
            pltpu.VMEM((4, BS), jnp.int32),
            pltpu.VMEM((BS, D), jnp.float32),
            pltpu.VMEM((BS, D), jnp.float32),
            pltpu.VMEM_SHARED((NP, D), jnp.float32),
        ] + [pltpu.SemaphoreType.DMA] * 8,
    )
    def agg(src_hbm, dst_hbm, t_hbm, zeros_hbm, out0, out1,
            idx_s, didx, b0, b1, acc,
            gs0, gs1, ss0, ss1, is0, is1, is2, is3):
        bufs = (b0, b1)
        gsem = (gs0, gs1)
        ssem = (ss0, ss1)
        isem = (is0, is1, is2, is3)
        c = lax.axis_index("c")
        s = lax.axis_index("s")
        w = c * 16 + s
        rs = s * RB

        def pf_dst(q, jj):
            pltpu.async_copy(dst_hbm.at[w, jj], didx.at[q], isem[q])

        def pf_dst_wait(q):
            pltpu.make_async_copy(dst_hbm.at[w, 0], didx.at[q], isem[q]).wait()

        def gstart(u, jj):
            pltpu.async_copy(t_hbm.at[idx_s.at[jj]], bufs[u], gsem[u])

        def gwait(u):
            pltpu.make_async_copy(t_hbm.at[idx_s.at[0]], bufs[u], gsem[u]).wait()

        def sstart(u, q):
            pltpu.async_copy(bufs[u], acc.at[didx.at[q]], ssem[u], add=True)

        def swait(u):
            pltpu.make_async_copy(bufs[u], acc.at[didx.at[0]], ssem[u]).wait()

        pltpu.sync_copy(src_hbm.at[w], idx_s)

        @pl.when(c == 0)
        def _():
            pltpu.sync_copy(t_hbm.at[pl.ds(rs, RB)], acc.at[pl.ds(rs, RB)])

        @pl.when(c != 0)
        def _():
            pltpu.sync_copy(zeros_hbm.at[pl.ds(rs, RB)], acc.at[pl.ds(rs, RB)])

        plsc.subcore_barrier()

        pf_dst(0, 0)
        pf_dst(1, 1)
        pf_dst(2, 2)
        gstart(0, 0)

        def step(j, u, q, first):
            gwait(u)
            pf_dst_wait(q)
            sstart(u, q)
            if not first:
                swait(1 - u)
            gstart(1 - u, lax.rem(j + 1, NB))
            pf_dst((q + 3) % 4, lax.rem(j + 3, NB))

        step(0, 0, 0, True)
        step(1, 1, 1, False)
        step(2, 0, 2, False)
        step(3, 1, 3, False)

        def body(i, carry):
            j0 = 4 * i
            step(j0, 0, 0, False)
            step(j0 + 1, 1, 1, False)
            step(j0 + 2, 0, 2, False)
            step(j0 + 3, 1, 3, False)
            return carry

        lax.fori_loop(1, NB // 4, body, 0)

        swait(1)
        gwait(0)
        pf_dst_wait(0)
        pf_dst_wait(1)
        pf_dst_wait(2)
        plsc.subcore_barrier()

        @pl.when(c == 0)
        def _():
            pltpu.sync_copy(acc.at[pl.ds(rs, RB)], out0.at[pl.ds(rs, RB)])

        @pl.when(c != 0)
        def _():
            pltpu.sync_copy(acc.at[pl.ds(rs, RB)], out1.at[pl.ds(rs, RB)])

    return agg


_agg128 = _make_agg(128)


def _first_body(x_ref, w_ref, d0_ref, d1_ref, t_ref, dinv_ref):
    deg = d0_ref[...] + d1_ref[...]
    dinv = lax.rsqrt(deg)
    mm = lax.dot_general(x_ref[...], w_ref[...], (((1,), (0,)), ((), ())),
                         precision=lax.Precision.HIGHEST,
                         preferred_element_type=jnp.float32)
    t_ref[...] = mm * dinv
    dinv_ref[...] = dinv


def _first(xp, W1, d0, d1):
    return pl.pallas_call(
        _first_body,
        grid=(16,),
        in_specs=[
            pl.BlockSpec((RB, 128), lambda i: (i, 0)),
            pl.BlockSpec((128, 128), lambda i: (0, 0)),
            pl.BlockSpec((RB, 1), lambda i: (i, 0)),
            pl.BlockSpec((RB, 1), lambda i: (i, 0)),
        ],
        out_specs=[
            pl.BlockSpec((RB, 128), lambda i: (i, 0)),
            pl.BlockSpec((RB, 1), lambda i: (i, 0)),
        ],
        out_shape=[
            jax.ShapeDtypeStruct((NP, 128), jnp.float32),
            jax.ShapeDtypeStruct((NP, 1), jnp.float32),
        ],
    )(xp, W1, d0, d1)


def _mid_body(a0_ref, a1_ref, dinv_ref, b_ref, w_ref, t_ref):
    dinv = dinv_ref[...]
    act = jnp.maximum((a0_ref[...] + a1_ref[...]) * dinv + b_ref[...], 0.0)
    mm = lax.dot_general(act, w_ref[...], (((1,), (0,)), ((), ())),
                         precision=lax.Precision.HIGHEST,
                         preferred_element_type=jnp.float32)
    t_ref[...] = mm * dinv


def _mid(a0, a1, dinv, b, W, d_out):
    return pl.pallas_call(
        _mid_body,
        grid=(16,),
        in_specs=[
            pl.BlockSpec((RB, 128), lambda i: (i, 0)),
            pl.BlockSpec((RB, 128), lambda i: (i, 0)),
            pl.BlockSpec((RB, 1), lambda i: (i, 0)),
            pl.BlockSpec((128,), lambda i: (0,)),
            pl.BlockSpec((128, d_out), lambda i: (0, 0)),
        ],
        out_specs=pl.BlockSpec((RB, d_out), lambda i: (i, 0)),
        out_shape=jax.ShapeDtypeStruct((NP, d_out), jnp.float32),
    )(a0, a1, dinv, b, W)


def _final_body(a0_ref, a1_ref, dinv_ref, b_ref, o_ref):
    a = a0_ref[...] + a1_ref[...]
    o_ref[...] = a[:, :40] * dinv_ref[...] + b_ref[...]


def _final(a0, a1, dinv, b3):
    return pl.pallas_call(
        _final_body,
        grid=(25,),
        in_specs=[
            pl.BlockSpec((400, 128), lambda i: (i, 0)),
            pl.BlockSpec((400, 128), lambda i: (i, 0)),
            pl.BlockSpec((400, 1), lambda i: (i, 0)),
            pl.BlockSpec((40,), lambda i: (0,)),
        ],
        out_specs=pl.BlockSpec((400, 40), lambda i: (i, 0)),
        out_shape=jax.ShapeDtypeStruct((N, 40), jnp.float32),
    )(a0, a1, dinv, b3)


def kernel(x, edge_index, W1, b1, W2, b2, W3, b3):
    x = x.astype(jnp.float32)
    src = edge_index[0].astype(jnp.int32)
    dst = edge_index[1].astype(jnp.int32)
    pad = jnp.full((EPAD - E,), PADROW, jnp.int32)
    src3 = jnp.concatenate([src, pad]).reshape(NT, NB, BS)
    dst3 = jnp.concatenate([dst, pad]).reshape(NT, NB, BS)

    xp = jnp.pad(x, ((0, NP - N), (0, 0)))
    zeros128 = jnp.zeros((NP, 128), jnp.float32)
    ones128 = jnp.ones((NP, 128), jnp.float32)
    W3p = jnp.pad(W3.astype(jnp.float32), ((0, 0), (0, 128 - 40)))

    g0, g1 = _agg128(src3, dst3, ones128, zeros128)
    t1, dinv = _first(xp, W1, g0[:, :1], g1[:, :1])
    a0, a1 = _agg128(src3, dst3, t1, zeros128)
    t2 = _mid(a0, a1, dinv, b1, W2, 128)
    a0, a1 = _agg128(src3, dst3, t2, zeros128)
    t3 = _mid(a0, a1, dinv, b2, W3p, 128)
    a0, a1 = _agg128(src3, dst3, t3, zeros128)
    return _final(a0, a1, dinv, b3)


# speedup vs baseline: 1.7317x; 1.1710x over previous
import functools

import jax
import jax.numpy as jnp
from jax import lax
from jax.experimental import pallas as pl
from jax.experimental.pallas import tpu as pltpu
from jax.experimental.pallas import tpu_sc as plsc

N = 10000
NP = 10112
PADROW = NP - 1
E = 320000
NT = 32
BS = 128
NB = 80
EPT = NB * BS
EPAD = EPT * NT
RB = NP // 16

_MESH = dict(core_axis_name="c", subcore_axis_name="s")


def _make_agg(D):
    @functools.partial(
        pl.kernel,
        mesh=plsc.VectorSubcoreMesh(**_MESH),
        out_type=[jax.ShapeDtypeStruct((NP, D), jnp.float32),
                  jax.ShapeDtypeStruct((NP, D), jnp.float32)],
        scratch_types=[
            pltpu.VMEM((NB, BS), jnp.int32),
            pltpu.VMEM((4, BS), jnp.int32),
            pltpu.VMEM((BS, D), jnp.float32),
            pltpu.VMEM((BS, D), jnp.float32),
            pltpu.VMEM_SHARED((NP, D), jnp.float32),
        ] + [pltpu.SemaphoreType.DMA] * 8,
    )
    def agg(src_hbm, dst_hbm, t_hbm, zeros_hbm, out0, out1,
            idx_s, didx, b0, b1, acc,
            gs0, gs1, ss0, ss1, is0, is1, is2, is3):
        bufs = (b0, b1)
        gsem = (gs0, gs1)
        ssem = (ss0, ss1)
        isem = (is0, is1, is2, is3)
        c = lax.axis_index("c")
        s = lax.axis_index("s")
        w = c * 16 + s
        rs = s * RB

        def pf_dst(q, jj):
            pltpu.async_copy(dst_hbm.at[w, jj], didx.at[q], isem[q])

        def pf_dst_wait(q):
            pltpu.make_async_copy(dst_hbm.at[w, 0], didx.at[q], isem[q]).wait()

        def gstart(u, jj):
            pltpu.async_copy(t_hbm.at[idx_s.at[jj]], bufs[u], gsem[u])

        def gwait(u):
            pltpu.make_async_copy(t_hbm.at[idx_s.at[0]], bufs[u], gsem[u]).wait()

        def sstart(u, q):
            pltpu.async_copy(bufs[u], acc.at[didx.at[q]], ssem[u], add=True)

        def swait(u):
            pltpu.make_async_copy(bufs[u], acc.at[didx.at[0]], ssem[u]).wait()

        pltpu.sync_copy(src_hbm.at[w], idx_s)

        @pl.when(c == 0)
        def _():
            pltpu.sync_copy(t_hbm.at[pl.ds(rs, RB)], acc.at[pl.ds(rs, RB)])

        @pl.when(c != 0)
        def _():
            pltpu.sync_copy(zeros_hbm.at[pl.ds(rs, RB)], acc.at[pl.ds(rs, RB)])

        plsc.subcore_barrier()

        pf_dst(0, 0)
        pf_dst(1, 1)
        pf_dst(2, 2)
        gstart(0, 0)

        def step(j, u, q, first):
            gwait(u)
            pf_dst_wait(q)
            sstart(u, q)
            if not first:
                swait(1 - u)
            gstart(1 - u, lax.rem(j + 1, NB))
            pf_dst((q + 3) % 4, lax.rem(j + 3, NB))

        step(0, 0, 0, True)
        step(1, 1, 1, False)
        step(2, 0, 2, False)
        step(3, 1, 3, False)

        def body(i, carry):
            j0 = 4 * i
            step(j0, 0, 0, False)
            step(j0 + 1, 1, 1, False)
            step(j0 + 2, 0, 2, False)
            step(j0 + 3, 1, 3, False)
            return carry

        lax.fori_loop(1, NB // 4, body, 0)

        swait(1)
        gwait(0)
        pf_dst_wait(0)
        pf_dst_wait(1)
        pf_dst_wait(2)
        plsc.subcore_barrier()

        @pl.when(c == 0)
        def _():
            pltpu.sync_copy(acc.at[pl.ds(rs, RB)], out0.at[pl.ds(rs, RB)])

        @pl.when(c != 0)
        def _():
            pltpu.sync_copy(acc.at[pl.ds(rs, RB)], out1.at[pl.ds(rs, RB)])

    return agg


_agg128 = _make_agg(128)


DW = 128


@functools.partial(
    pl.kernel,
    mesh=plsc.VectorSubcoreMesh(**_MESH),
    out_type=[jax.ShapeDtypeStruct((NP, DW), jnp.float32),
              jax.ShapeDtypeStruct((NP, DW), jnp.float32)],
    scratch_types=[
        pltpu.VMEM((NB, BS), jnp.int32),
        pltpu.VMEM((BS, DW), jnp.float32),
        pltpu.VMEM_SHARED((NP, DW), jnp.float32),
        pltpu.SemaphoreType.DMA,
    ],
)
def _deg(dst_hbm, ones_hbm, zeros_hbm, out0, out1, idx_d, onesb, acc, sem0):
    c = lax.axis_index("c")
    s = lax.axis_index("s")
    w = c * 16 + s
    rs = s * RB
    pltpu.sync_copy(dst_hbm.at[w], idx_d)
    pltpu.sync_copy(ones_hbm.at[pl.ds(0, BS)], onesb)

    @pl.when(c == 0)
    def _():
        pltpu.sync_copy(ones_hbm.at[pl.ds(rs, RB)], acc.at[pl.ds(rs, RB)])

    @pl.when(c != 0)
    def _():
        pltpu.sync_copy(zeros_hbm.at[pl.ds(rs, RB)], acc.at[pl.ds(rs, RB)])

    plsc.subcore_barrier()

    def body(j, carry):
        pltpu.sync_copy(onesb, acc.at[idx_d.at[j]], add=True)
        return carry

    lax.fori_loop(0, NB, body, 0)
    plsc.subcore_barrier()

    @pl.when(c == 0)
    def _():
        pltpu.sync_copy(acc.at[pl.ds(rs, RB)], out0.at[pl.ds(rs, RB)])

    @pl.when(c != 0)
    def _():
        pltpu.sync_copy(acc.at[pl.ds(rs, RB)], out1.at[pl.ds(rs, RB)])


def _first_body(x_ref, w_ref, d0_ref, d1_ref, t_ref, dinv_ref):
    deg = d0_ref[...] + d1_ref[...]
    dinv = lax.rsqrt(deg)
    mm = lax.dot_general(x_ref[...], w_ref[...], (((1,), (0,)), ((), ())),
                         precision=lax.Precision.HIGHEST,
                         preferred_element_type=jnp.float32)
    t_ref[...] = mm * dinv
    dinv_ref[...] = dinv


def _first(xp, W1, d0, d1):
    return pl.pallas_call(
        _first_body,
        grid=(16,),
        in_specs=[
            pl.BlockSpec((RB, 128), lambda i: (i, 0)),
            pl.BlockSpec((128, 128), lambda i: (0, 0)),
            pl.BlockSpec((RB, 1), lambda i: (i, 0)),
            pl.BlockSpec((RB, 1), lambda i: (i, 0)),
        ],
        out_specs=[
            pl.BlockSpec((RB, 128), lambda i: (i, 0)),
            pl.BlockSpec((RB, 1), lambda i: (i, 0)),
        ],
        out_shape=[
            jax.ShapeDtypeStruct((NP, 128), jnp.float32),
            jax.ShapeDtypeStruct((NP, 1), jnp.float32),
        ],
    )(xp, W1, d0, d1)


def _mid_body(a0_ref, a1_ref, dinv_ref, b_ref, w_ref, t_ref):
    dinv = dinv_ref[...]
    act = jnp.maximum((a0_ref[...] + a1_ref[...]) * dinv + b_ref[...], 0.0)
    mm = lax.dot_general(act, w_ref[...], (((1,), (0,)), ((), ())),
                         precision=lax.Precision.HIGHEST,
                         preferred_element_type=jnp.float32)
    t_ref[...] = mm * dinv


def _mid(a0, a1, dinv, b, W, d_out):
    return pl.pallas_call(
        _mid_body,
        grid=(16,),
        in_specs=[
            pl.BlockSpec((RB, 128), lambda i: (i, 0)),
            pl.BlockSpec((RB, 128), lambda i: (i, 0)),
            pl.BlockSpec((RB, 1), lambda i: (i, 0)),
            pl.BlockSpec((128,), lambda i: (0,)),
            pl.BlockSpec((128, d_out), lambda i: (0, 0)),
        ],
        out_specs=pl.BlockSpec((RB, d_out), lambda i: (i, 0)),
        out_shape=jax.ShapeDtypeStruct((NP, d_out), jnp.float32),
    )(a0, a1, dinv, b, W)


def _final_body(a0_ref, a1_ref, dinv_ref, b_ref, o_ref):
    a = a0_ref[...] + a1_ref[...]
    o_ref[...] = a[:, :40] * dinv_ref[...] + b_ref[...]


def _final(a0, a1, dinv, b3):
    return pl.pallas_call(
        _final_body,
        grid=(25,),
        in_specs=[
            pl.BlockSpec((400, 128), lambda i: (i, 0)),
            pl.BlockSpec((400, 128), lambda i: (i, 0)),
            pl.BlockSpec((400, 1), lambda i: (i, 0)),
            pl.BlockSpec((40,), lambda i: (0,)),
        ],
        out_specs=pl.BlockSpec((400, 40), lambda i: (i, 0)),
        out_shape=jax.ShapeDtypeStruct((N, 40), jnp.float32),
    )(a0, a1, dinv, b3)


def kernel(x, edge_index, W1, b1, W2, b2, W3, b3):
    x = x.astype(jnp.float32)
    src = edge_index[0].astype(jnp.int32)
    dst = edge_index[1].astype(jnp.int32)
    pad = jnp.full((EPAD - E,), PADROW, jnp.int32)
    src3 = jnp.concatenate([src, pad]).reshape(NT, NB, BS)
    dst3 = jnp.concatenate([dst, pad]).reshape(NT, NB, BS)

    xp = jnp.pad(x, ((0, NP - N), (0, 0)))
    zeros128 = jnp.zeros((NP, 128), jnp.float32)
    ones128 = jnp.ones((NP, 128), jnp.float32)
    W3p = jnp.pad(W3.astype(jnp.float32), ((0, 0), (0, 128 - 40)))

    g0, g1 = _deg(dst3, ones128, zeros128)
    t1, dinv = _first(xp, W1, g0[:, :1], g1[:, :1])
    a0, a1 = _agg128(src3, dst3, t1, zeros128)
    t2 = _mid(a0, a1, dinv, b1, W2, 128)
    a0, a1 = _agg128(src3, dst3, t2, zeros128)
    t3 = _mid(a0, a1, dinv, b2, W3p, 128)
    a0, a1 = _agg128(src3, dst3, t3, zeros128)
    return _final(a0, a1, dinv, b3)
